# baseline (speedup 1.0000x reference)
import jax
import jax.numpy as jnp
from jax import lax
from jax.experimental import pallas as pl
from jax.experimental.pallas import tpu as pltpu
from jax.experimental.pallas import tpu_sc as plsc

N_NODES = 10000
SEQ = 32
SEQH = SEQ // 2
CH = 128
M_PAD = 10240

TC_BM = 2000
TC_GRID = N_NODES // TC_BM

NW = 32
NODES_PER_W = M_PAD // NW
NODES_PER_CHUNK = 8
CHUNKS = NODES_PER_W // NODES_PER_CHUNK
IDX_PER_CHUNK = NODES_PER_CHUNK * SEQH
NBUF = 5


def _zk_body(x_ref, w_ref, o_ref):
    acc = lax.dot_general(
        x_ref[...], w_ref[...], (((1,), (0,)), ((), ())),
        preferred_element_type=jnp.float32)
    for j in range(SEQH):
        o_ref[:, j, :] = acc[:, CH * j:CH * (j + 1)]


def _z_transform(x, w4h):
    return pl.pallas_call(
        _zk_body,
        grid=(TC_GRID,),
        in_specs=[
            pl.BlockSpec((TC_BM, CH), lambda i: (i, 0)),
            pl.BlockSpec((CH, SEQH * CH), lambda i: (0, 0)),
        ],
        out_specs=pl.BlockSpec((TC_BM, SEQH, CH), lambda i: (i, 0, 0)),
        out_shape=jax.ShapeDtypeStruct((N_NODES, SEQH, CH), jnp.float32),
    )(x, w4h)


def _make_bag(seed_is_bias):
    def body(z_ref, idx_ref, seed_ref, o_ref, idxv, bv, gbuf, sbuf, obuf,
             semg, sems, semo):
        cid = lax.axis_index("c")
        sid = lax.axis_index("s")
        wid = (1 - cid) * 16 + sid

        pltpu.sync_copy(idx_ref.at[wid], idxv)
        if seed_is_bias:
            pltpu.sync_copy(seed_ref, bv)
        bregs = [bv[pl.ds(16 * g, 16)] for g in range(8)]

        def seed_rows(c):
            return seed_ref.at[pl.ds(wid * NODES_PER_W + c * NODES_PER_CHUNK,
                                     NODES_PER_CHUNK)]

        def out_rows(c):
            return o_ref.at[pl.ds(wid * NODES_PER_W + c * NODES_PER_CHUNK,
                                  NODES_PER_CHUNK)]

        def start_gather(c, buf):
            pltpu.async_copy(z_ref.at[idxv.at[c]], gbuf.at[buf], semg.at[buf])
            if not seed_is_bias:
                pltpu.async_copy(seed_rows(c), sbuf.at[buf], sems.at[buf])

        def wait_gather(c, buf):
            pltpu.make_async_copy(z_ref.at[idxv.at[c]], gbuf.at[buf],
                                  semg.at[buf]).wait()
            if not seed_is_bias:
                pltpu.make_async_copy(seed_rows(c), sbuf.at[buf],
                                      sems.at[buf]).wait()

        LEAD = NBUF - 2
        for c0 in range(LEAD):
            start_gather(c0, c0)

        def step(i, _):
            for u in range(NBUF):
                c = i * NBUF + u

                @pl.when(c + LEAD < CHUNKS)
                def _():
                    start_gather(c + LEAD, (u + LEAD) % NBUF)

                wait_gather(c, u)

                p2 = u

                @pl.when(c >= NBUF)
                def _():
                    pltpu.make_async_copy(obuf.at[p2], out_rows(c),
                                          semo.at[p2]).wait()

                def node_body(q, _):
                    if seed_is_bias:
                        a = list(bregs)
                    else:
                        a = [sbuf[u, q, pl.ds(16 * g, 16)] for g in range(8)]
                    for r in range(SEQH):
                        for g in range(8):
                            a[g] = a[g] + gbuf[u, q * SEQH + r,
                                               pl.ds(16 * g, 16)]
                    for g in range(8):
                        obuf[p2, q, pl.ds(16 * g, 16)] = a[g]
                    return _

                lax.fori_loop(0, NODES_PER_CHUNK, node_body, None)
                pltpu.async_copy(obuf.at[p2], out_rows(c), semo.at[p2])
            return _

        lax.fori_loop(0, CHUNKS // NBUF, step, None)

        for p2 in range(NBUF):
            c = CHUNKS - NBUF + p2
            pltpu.make_async_copy(obuf.at[p2], out_rows(c),
                                  semo.at[p2]).wait()

    return pl.kernel(
        body,
        out_type=jax.ShapeDtypeStruct((M_PAD, CH), jnp.float32),
        mesh=plsc.VectorSubcoreMesh(core_axis_name="c", subcore_axis_name="s"),
        scratch_types=[
            pltpu.VMEM((CHUNKS, IDX_PER_CHUNK), jnp.int32),
            pltpu.VMEM((CH,), jnp.float32),
            pltpu.VMEM((NBUF, IDX_PER_CHUNK, CH), jnp.float32),
            pltpu.VMEM((NBUF, NODES_PER_CHUNK, CH), jnp.float32),
            pltpu.VMEM((NBUF, NODES_PER_CHUNK, CH), jnp.float32),
            pltpu.SemaphoreType.DMA((NBUF,)),
            pltpu.SemaphoreType.DMA((NBUF,)),
            pltpu.SemaphoreType.DMA((NBUF,)),
        ],
    )


_bag_a = _make_bag(seed_is_bias=True)
_bag_b = _make_bag(seed_is_bias=False)


def kernel(x, indices, W, b):
    idx32 = indices.astype(jnp.int32)
    jj = jnp.arange(SEQH, dtype=jnp.int32)[None, :]
    flat_a = idx32[:, :SEQH] * SEQH + jj
    flat_b = idx32[:, SEQH:] * SEQH + jj
    n_pad = M_PAD - N_NODES
    pad_rows = (jnp.arange(n_pad * SEQH, dtype=jnp.int32)
                .reshape(n_pad, SEQH) * 997) % (N_NODES * SEQH)
    flat_a = jnp.concatenate([flat_a, pad_rows], axis=0)
    flat_a = flat_a.reshape(NW, CHUNKS, IDX_PER_CHUNK)
    flat_b = jnp.concatenate([flat_b, pad_rows], axis=0)
    flat_b = flat_b.reshape(NW, CHUNKS, IDX_PER_CHUNK)

    w4 = W.reshape(CH, SEQ, CH).transpose(2, 1, 0).reshape(CH, SEQ * CH)

    za = _z_transform(x, w4[:, :SEQH * CH])
    zb = _z_transform(x, w4[:, SEQH * CH:])
    pa = _bag_a(za.reshape(N_NODES * SEQH, CH), flat_a, b)
    out = _bag_b(zb.reshape(N_NODES * SEQH, CH), flat_b, pa)
    return out[:N_NODES]


# speedup vs baseline: 1.0055x; 1.0055x over previous
import jax
import jax.numpy as jnp
from jax import lax
from jax.experimental import pallas as pl
from jax.experimental.pallas import tpu as pltpu
from jax.experimental.pallas import tpu_sc as plsc

N_NODES = 10000
SEQ = 32
SEQH = SEQ // 2
CH = 128
M_PAD = 10240

TC_BM = 1000
TC_GRID = N_NODES // TC_BM

NW = 32
NODES_PER_W = M_PAD // NW
NODES_PER_CHUNK = 8
CHUNKS = NODES_PER_W // NODES_PER_CHUNK
IDX_PER_CHUNK = NODES_PER_CHUNK * SEQH
NBUF = 5


def _zk_body(x_ref, w_ref, o_ref):
    acc = lax.dot_general(
        x_ref[...], w_ref[...], (((1,), (0,)), ((), ())),
        preferred_element_type=jnp.float32)
    for j in range(SEQH):
        o_ref[:, j, :] = acc[:, CH * j:CH * (j + 1)]


def _z_transform(x, w4h):
    return pl.pallas_call(
        _zk_body,
        grid=(TC_GRID,),
        in_specs=[
            pl.BlockSpec((TC_BM, CH), lambda i: (i, 0)),
            pl.BlockSpec((CH, SEQH * CH), lambda i: (0, 0)),
        ],
        out_specs=pl.BlockSpec((TC_BM, SEQH, CH), lambda i: (i, 0, 0)),
        out_shape=jax.ShapeDtypeStruct((N_NODES, SEQH, CH), jnp.float32),
    )(x, w4h)


def _make_bag(seed_is_bias):
    def body(z_ref, idx_ref, seed_ref, o_ref, idxv, bv, gbuf, sbuf, obuf,
             semg, sems, semo):
        cid = lax.axis_index("c")
        sid = lax.axis_index("s")
        wid = (1 - cid) * 16 + sid

        pltpu.sync_copy(idx_ref.at[wid], idxv)
        if seed_is_bias:
            pltpu.sync_copy(seed_ref, bv)
        bregs = [bv[pl.ds(16 * g, 16)] for g in range(8)]

        def seed_rows(c):
            return seed_ref.at[pl.ds(wid * NODES_PER_W + c * NODES_PER_CHUNK,
                                     NODES_PER_CHUNK)]

        def out_rows(c):
            return o_ref.at[pl.ds(wid * NODES_PER_W + c * NODES_PER_CHUNK,
                                  NODES_PER_CHUNK)]

        def start_gather(c, buf):
            pltpu.async_copy(z_ref.at[idxv.at[c]], gbuf.at[buf], semg.at[buf])
            if not seed_is_bias:
                pltpu.async_copy(seed_rows(c), sbuf.at[buf], sems.at[buf])

        def wait_gather(c, buf):
            pltpu.make_async_copy(z_ref.at[idxv.at[c]], gbuf.at[buf],
                                  semg.at[buf]).wait()
            if not seed_is_bias:
                pltpu.make_async_copy(seed_rows(c), sbuf.at[buf],
                                      sems.at[buf]).wait()

        LEAD = NBUF - 2
        for c0 in range(LEAD):
            start_gather(c0, c0)

        def step(i, _):
            for u in range(NBUF):
                c = i * NBUF + u

                @pl.when(c + LEAD < CHUNKS)
                def _():
                    start_gather(c + LEAD, (u + LEAD) % NBUF)

                wait_gather(c, u)

                p2 = u

                @pl.when(c >= NBUF)
                def _():
                    pltpu.make_async_copy(obuf.at[p2], out_rows(c),
                                          semo.at[p2]).wait()

                def node_body(q, _):
                    if seed_is_bias:
                        a = list(bregs)
                    else:
                        a = [sbuf[u, q, pl.ds(16 * g, 16)] for g in range(8)]
                    for r in range(SEQH):
                        for g in range(8):
                            a[g] = a[g] + gbuf[u, q * SEQH + r,
                                               pl.ds(16 * g, 16)]
                    for g in range(8):
                        obuf[p2, q, pl.ds(16 * g, 16)] = a[g]
                    return _

                lax.fori_loop(0, NODES_PER_CHUNK, node_body, None)
                pltpu.async_copy(obuf.at[p2], out_rows(c), semo.at[p2])
            return _

        lax.fori_loop(0, CHUNKS // NBUF, step, None)

        for p2 in range(NBUF):
            c = CHUNKS - NBUF + p2
            pltpu.make_async_copy(obuf.at[p2], out_rows(c),
                                  semo.at[p2]).wait()

    return pl.kernel(
        body,
        out_type=jax.ShapeDtypeStruct((M_PAD, CH), jnp.float32),
        mesh=plsc.VectorSubcoreMesh(core_axis_name="c", subcore_axis_name="s"),
        scratch_types=[
            pltpu.VMEM((CHUNKS, IDX_PER_CHUNK), jnp.int32),
            pltpu.VMEM((CH,), jnp.float32),
            pltpu.VMEM((NBUF, IDX_PER_CHUNK, CH), jnp.float32),
            pltpu.VMEM((NBUF, NODES_PER_CHUNK, CH), jnp.float32),
            pltpu.VMEM((NBUF, NODES_PER_CHUNK, CH), jnp.float32),
            pltpu.SemaphoreType.DMA((NBUF,)),
            pltpu.SemaphoreType.DMA((NBUF,)),
            pltpu.SemaphoreType.DMA((NBUF,)),
        ],
    )


_bag_a = _make_bag(seed_is_bias=True)
_bag_b = _make_bag(seed_is_bias=False)


def kernel(x, indices, W, b):
    idx32 = indices.astype(jnp.int32)
    jj = jnp.arange(SEQH, dtype=jnp.int32)[None, :]
    flat_a = idx32[:, :SEQH] * SEQH + jj
    flat_b = idx32[:, SEQH:] * SEQH + jj
    n_pad = M_PAD - N_NODES
    pad_rows = (jnp.arange(n_pad * SEQH, dtype=jnp.int32)
                .reshape(n_pad, SEQH) * 997) % (N_NODES * SEQH)
    flat_a = jnp.concatenate([flat_a, pad_rows], axis=0)
    flat_a = flat_a.reshape(NW, CHUNKS, IDX_PER_CHUNK)
    flat_b = jnp.concatenate([flat_b, pad_rows], axis=0)
    flat_b = flat_b.reshape(NW, CHUNKS, IDX_PER_CHUNK)

    w4 = W.reshape(CH, SEQ, CH).transpose(2, 1, 0).reshape(CH, SEQ * CH)

    za = _z_transform(x, w4[:, :SEQH * CH])
    zb = _z_transform(x, w4[:, SEQH * CH:])
    pa = _bag_a(za.reshape(N_NODES * SEQH, CH), flat_a, b)
    out = _bag_b(zb.reshape(N_NODES * SEQH, CH), flat_b, pa)
    return out[:N_NODES]
